# Initial kernel scaffold; baseline (speedup 1.0000x reference)
import jax
import jax.numpy as jnp
from jax.experimental import pallas as pl


def kernel(inputs, embedding):
    raise NotImplementedError("write your pallas kernel here")


# speedup vs baseline: 2.7603x; 2.7603x over previous
import functools

import jax
import jax.numpy as jnp
from jax import lax
from jax.experimental import pallas as pl
from jax.experimental.pallas import tpu as pltpu
from jax.experimental.pallas import tpu_sc as plsc

_D = 128
_C = 128


@functools.lru_cache(maxsize=None)
def _build_gather(B: int, D: int):
    info = plsc.get_sparse_core_info()
    nc, ns = info.num_cores, info.num_subcores
    nw = nc * ns
    b_per_w = B // nw
    n_chunks = b_per_w // _C
    assert b_per_w % _C == 0

    mesh = plsc.VectorSubcoreMesh(core_axis_name="c", subcore_axis_name="s")

    def body(table_hbm, idx_hbm, out_hbm, idx_v, rows_v, sem):
        wid = lax.axis_index("s") * nc + lax.axis_index("c")
        base = wid * b_per_w

        def chunk(j, carry):
            start = base + j * _C
            pltpu.sync_copy(idx_hbm.at[pl.ds(start, _C)], idx_v)
            pltpu.async_copy(table_hbm.at[idx_v], rows_v, sem).wait()
            pltpu.sync_copy(rows_v, out_hbm.at[pl.ds(start, _C)])
            return carry

        lax.fori_loop(0, n_chunks, chunk, 0)

    return pl.kernel(
        body,
        mesh=mesh,
        out_type=jax.ShapeDtypeStruct((B, D), jnp.float32),
        scratch_types=[
            pltpu.VMEM((_C,), jnp.int32),
            pltpu.VMEM((_C, D), jnp.float32),
            pltpu.SemaphoreType.DMA,
        ],
    )


def kernel(inputs, embedding):
    batch, steps = inputs.shape
    b = batch * steps
    idx = inputs.reshape(b).astype(jnp.int32)
    out = _build_gather(b, embedding.shape[1])(embedding, idx)
    return out.reshape(batch, steps, embedding.shape[1])


# speedup vs baseline: 3.3304x; 1.2065x over previous
import functools

import jax
import jax.numpy as jnp
from jax import lax
from jax.experimental import pallas as pl
from jax.experimental.pallas import tpu as pltpu
from jax.experimental.pallas import tpu_sc as plsc

_C = 128
_NBUF = 5


@functools.lru_cache(maxsize=None)
def _build_gather(B: int, D: int):
    info = plsc.get_sparse_core_info()
    nc, ns = info.num_cores, info.num_subcores
    nw = nc * ns
    b_per_w = B // nw
    n_chunks = b_per_w // _C
    n_periods = n_chunks // _NBUF
    assert b_per_w % _C == 0 and n_chunks % _NBUF == 0
    total_chunks = B // _C

    mesh = plsc.VectorSubcoreMesh(core_axis_name="c", subcore_axis_name="s")

    def body(table_hbm, idx_hbm, out_hbm, idx_v, rows_v, gsem, *wsems):
        wid = lax.axis_index("s") * nc + lax.axis_index("c")
        cbase = wid * n_chunks
        pltpu.sync_copy(idx_hbm.at[wid], idx_v)

        def period(o, carry):
            descs = []
            for b in range(_NBUF):
                j = o * _NBUF + b

                @pl.when(o > 0)
                def _wait_writeout():
                    pltpu.make_async_copy(
                        rows_v.at[b], out_hbm.at[cbase], wsems[b]).wait()

                descs.append(pltpu.async_copy(
                    table_hbm.at[idx_v.at[j]], rows_v.at[b], gsem))
            for b in range(_NBUF):
                j = o * _NBUF + b
                descs[b].wait()
                pltpu.async_copy(rows_v.at[b], out_hbm.at[cbase + j], wsems[b])
            return carry

        lax.fori_loop(0, n_periods, period, 0)
        for b in range(_NBUF):
            pltpu.make_async_copy(
                rows_v.at[b], out_hbm.at[cbase], wsems[b]).wait()

    return pl.kernel(
        body,
        mesh=mesh,
        out_type=jax.ShapeDtypeStruct((total_chunks, _C, D), jnp.float32),
        scratch_types=[
            pltpu.VMEM((n_chunks, _C), jnp.int32),
            pltpu.VMEM((_NBUF, _C, D), jnp.float32),
            pltpu.SemaphoreType.DMA,
        ] + [pltpu.SemaphoreType.DMA] * _NBUF,
    )


def kernel(inputs, embedding):
    batch, steps = inputs.shape
    b = batch * steps
    d = embedding.shape[1]
    info = plsc.get_sparse_core_info()
    nw = info.num_cores * info.num_subcores
    n_chunks = b // (nw * _C)
    idx = inputs.reshape(nw, n_chunks, _C).astype(jnp.int32)
    out = _build_gather(b, d)(embedding, idx)
    return out.reshape(batch, steps, d)


# speedup vs baseline: 5.9697x; 1.7925x over previous
import functools

import jax
import jax.numpy as jnp
from jax import lax
from jax.experimental import pallas as pl
from jax.experimental.pallas import tpu as pltpu
from jax.experimental.pallas import tpu_sc as plsc

_EPC = 2
_NBUF = 4


@functools.lru_cache(maxsize=None)
def _build_gather(batch: int, steps: int, D: int):
    info = plsc.get_sparse_core_info()
    nc, ns = info.num_cores, info.num_subcores
    nw = nc * ns
    e_per_w = batch // nw
    C = _EPC * steps
    n_chunks = e_per_w // _EPC
    n_periods = n_chunks // _NBUF
    assert C <= 128 and batch % nw == 0 and e_per_w % _EPC == 0
    assert n_chunks % _NBUF == 0

    mesh = plsc.VectorSubcoreMesh(core_axis_name="c", subcore_axis_name="s")

    def body(table_hbm, idx_hbm, out_hbm, idx_v, rows_v, gsem, *wsems):
        wid = lax.axis_index("s") * nc + lax.axis_index("c")
        ebase = wid * e_per_w
        pltpu.sync_copy(idx_hbm.at[wid], idx_v)

        def wait_writeout(b):
            for e in range(_EPC):
                pltpu.make_async_copy(
                    rows_v.at[b, pl.ds(e * steps, steps)],
                    out_hbm.at[ebase], wsems[b]).wait()

        def period(o, carry):
            descs = []
            for b in range(_NBUF):
                j = o * _NBUF + b

                @pl.when(o > 0)
                def _():
                    wait_writeout(b)

                descs.append(pltpu.async_copy(
                    table_hbm.at[idx_v.at[j]], rows_v.at[b], gsem))
            for b in range(_NBUF):
                j = o * _NBUF + b
                descs[b].wait()
                for e in range(_EPC):
                    pltpu.async_copy(
                        rows_v.at[b, pl.ds(e * steps, steps)],
                        out_hbm.at[ebase + j * _EPC + e], wsems[b])
            return carry

        lax.fori_loop(0, n_periods, period, 0)
        for b in range(_NBUF):
            wait_writeout(b)

    return pl.kernel(
        body,
        mesh=mesh,
        out_type=jax.ShapeDtypeStruct((batch, steps, D), jnp.float32),
        scratch_types=[
            pltpu.VMEM((n_chunks, C), jnp.int32),
            pltpu.VMEM((_NBUF, C, D), jnp.float32),
            pltpu.SemaphoreType.DMA,
        ] + [pltpu.SemaphoreType.DMA] * _NBUF,
    )


def kernel(inputs, embedding):
    batch, steps = inputs.shape
    d = embedding.shape[1]
    info = plsc.get_sparse_core_info()
    nw = info.num_cores * info.num_subcores
    n_chunks = batch // (nw * _EPC)
    idx = inputs.astype(jnp.int32).reshape(nw, n_chunks, _EPC * steps)
    return _build_gather(batch, steps, d)(embedding, idx)


# speedup vs baseline: 5.9747x; 1.0008x over previous
import functools

import jax
import jax.numpy as jnp
from jax import lax
from jax.experimental import pallas as pl
from jax.experimental.pallas import tpu as pltpu
from jax.experimental.pallas import tpu_sc as plsc

_EPC = 2
_NBUF = 4


@functools.lru_cache(maxsize=None)
def _build_gather(batch: int, steps: int, D: int):
    info = plsc.get_sparse_core_info()
    nc, ns = info.num_cores, info.num_subcores
    nw = nc * ns
    e_per_w = batch // nw
    C = _EPC * steps
    n_chunks = e_per_w // _EPC
    n_periods = n_chunks // _NBUF
    assert C <= 128 and batch % nw == 0 and e_per_w % _EPC == 0
    assert n_chunks % _NBUF == 0

    mesh = plsc.VectorSubcoreMesh(core_axis_name="c", subcore_axis_name="s")

    def body(table_hbm, idx_hbm, out_hbm, idx_v, rows_v, gsem, *wsems):
        wid = lax.axis_index("s") * nc + lax.axis_index("c")
        ebase = wid * e_per_w
        pltpu.sync_copy(idx_hbm.at[wid], idx_v)

        def wait_writeout(b):
            for e in range(_EPC):
                pltpu.make_async_copy(
                    rows_v.at[b, pl.ds(e * steps, steps)],
                    out_hbm.at[ebase], wsems[b]).wait()

        def period(o, carry):
            descs = []
            for b in range(_NBUF):
                j = o * _NBUF + b

                @pl.when(o > 0)
                def _():
                    wait_writeout(b)

                descs.append(pltpu.async_copy(
                    table_hbm.at[idx_v.at[j]], rows_v.at[b], gsem))
            for b in range(_NBUF):
                j = o * _NBUF + b
                descs[b].wait()
                for e in range(_EPC):
                    pltpu.async_copy(
                        rows_v.at[b, pl.ds(e * steps, steps)],
                        out_hbm.at[ebase + j * _EPC + e], wsems[b])
            return carry

        lax.fori_loop(0, n_periods, period, 0)
        for b in range(_NBUF):
            wait_writeout(b)

    return pl.kernel(
        body,
        mesh=mesh,
        compiler_params=pltpu.CompilerParams(use_tc_tiling_on_sc=True),
        out_type=jax.ShapeDtypeStruct((batch, steps, D), jnp.float32),
        scratch_types=[
            pltpu.VMEM((n_chunks, C), jnp.int32),
            pltpu.VMEM((_NBUF, C, D), jnp.float32),
            pltpu.SemaphoreType.DMA,
        ] + [pltpu.SemaphoreType.DMA] * _NBUF,
    )


def kernel(inputs, embedding):
    batch, steps = inputs.shape
    d = embedding.shape[1]
    info = plsc.get_sparse_core_info()
    nw = info.num_cores * info.num_subcores
    n_chunks = batch // (nw * _EPC)
    idx = inputs.astype(jnp.int32).reshape(nw, n_chunks, _EPC * steps)
    return _build_gather(batch, steps, d)(embedding, idx)
